# Initial kernel scaffold; baseline (speedup 1.0000x reference)
import jax
import jax.numpy as jnp
from jax.experimental import pallas as pl


def kernel(numeric, cat1, cat2, cat3, W1, b1, W2, b2, E1, E2, E3, W3, b3):
    raise NotImplementedError("write your pallas kernel here")


# speedup vs baseline: 6.7900x; 6.7900x over previous
import functools

import jax
import jax.numpy as jnp
from jax import lax
from jax.experimental import pallas as pl
from jax.experimental.pallas import tpu as pltpu
from jax.experimental.pallas import tpu_sc as plsc

B = 16384
D3 = 256
_GATHER_CHUNK = 128


def _leaky(x):
    return jnp.where(x > 0, x, 0.01 * x)


def _sc_gather(table, idx):
    info = plsc.get_sparse_core_info()
    nw = info.num_cores * info.num_subcores
    b_per_w = B // nw
    n_chunks = b_per_w // _GATHER_CHUNK

    mesh = plsc.VectorSubcoreMesh(core_axis_name="c", subcore_axis_name="s")

    @functools.partial(
        pl.kernel,
        mesh=mesh,
        out_type=jax.ShapeDtypeStruct((B, D3), jnp.float32),
        scratch_types=[
            pltpu.VMEM((_GATHER_CHUNK,), jnp.int32),
            pltpu.VMEM((_GATHER_CHUNK, D3), jnp.float32),
            pltpu.SemaphoreType.DMA,
        ],
    )
    def gather_k(table_hbm, idx_hbm, out_hbm, idx_v, rows_v, sem):
        wid = lax.axis_index("s") * info.num_cores + lax.axis_index("c")
        base = wid * b_per_w
        for c in range(n_chunks):
            off = base + c * _GATHER_CHUNK
            pltpu.sync_copy(idx_hbm.at[pl.ds(off, _GATHER_CHUNK)], idx_v)
            pltpu.async_copy(table_hbm.at[idx_v], rows_v, sem).wait()
            pltpu.sync_copy(rows_v, out_hbm.at[pl.ds(off, _GATHER_CHUNK)])

    return gather_k(table, idx)


def _tc_body(num_ref, c1_ref, c2_ref, e3_ref,
             W1T_ref, b1_ref, W2T_ref, b2_ref,
             E1_ref, E2_ref, W3vT_ref, W3e1T_ref, W3e2T_ref, W3e3T_ref,
             b3_ref, out_ref):
    blk = num_ref.shape[0]
    x = num_ref[...]
    v = _leaky(jnp.dot(x, W1T_ref[...], preferred_element_type=jnp.float32)
               + b1_ref[...])
    v = _leaky(jnp.dot(v, W2T_ref[...], preferred_element_type=jnp.float32)
               + b2_ref[...])
    acc = jnp.dot(v, W3vT_ref[...], preferred_element_type=jnp.float32)
    acc += jnp.dot(e3_ref[...], W3e3T_ref[...],
                   preferred_element_type=jnp.float32)
    oh1 = (c1_ref[...] == lax.broadcasted_iota(jnp.int32, (blk, 4), 1)
           ).astype(jnp.float32)
    e1 = jnp.dot(oh1, E1_ref[...], preferred_element_type=jnp.float32)
    acc += jnp.dot(e1, W3e1T_ref[...], preferred_element_type=jnp.float32)
    oh2 = (c2_ref[...] == lax.broadcasted_iota(jnp.int32, (blk, 5), 1)
           ).astype(jnp.float32)
    e2 = jnp.dot(oh2, E2_ref[...], preferred_element_type=jnp.float32)
    acc += jnp.dot(e2, W3e2T_ref[...], preferred_element_type=jnp.float32)
    acc += b3_ref[...]
    out_ref[...] = _leaky(acc)


def _tc_fused(numeric, c1, c2, e3, W1T, b1, W2T, b2,
              E1, E2, W3vT, W3e1T, W3e2T, W3e3T, b3, blk=1024):
    grid = B // blk

    def full(shape):
        return pl.BlockSpec(shape, lambda i: (0, 0))

    return pl.pallas_call(
        _tc_body,
        grid=(grid,),
        in_specs=[
            pl.BlockSpec((blk, 3), lambda i: (i, 0)),
            pl.BlockSpec((blk, 1), lambda i: (i, 0)),
            pl.BlockSpec((blk, 1), lambda i: (i, 0)),
            pl.BlockSpec((blk, D3), lambda i: (i, 0)),
            full((3, 64)),
            full((1, 64)),
            full((64, 128)),
            full((1, 128)),
            full((4, 16)),
            full((5, 25)),
            full((128, 128)),
            full((16, 128)),
            full((25, 128)),
            full((D3, 128)),
            full((1, 128)),
        ],
        out_specs=pl.BlockSpec((blk, 128), lambda i: (i, 0)),
        out_shape=jax.ShapeDtypeStruct((B, 128), jnp.float32),
    )(numeric, c1, c2, e3, W1T, b1, W2T, b2,
      E1, E2, W3vT, W3e1T, W3e2T, W3e3T, b3)


def kernel(numeric, cat1, cat2, cat3, W1, b1, W2, b2, E1, E2, E3, W3, b3):
    idx = cat3.reshape(B).astype(jnp.int32)
    e3 = _sc_gather(E3, idx)

    c1 = cat1.reshape(B, 1).astype(jnp.int32)
    c2 = cat2.reshape(B, 1).astype(jnp.int32)
    W3vT = W3[:, 0:128].T
    W3e1T = W3[:, 128:144].T
    W3e2T = W3[:, 144:169].T
    W3e3T = W3[:, 169:425].T
    return _tc_fused(numeric, c1, c2, e3,
                     W1.T, b1.reshape(1, 64), W2.T, b2.reshape(1, 128),
                     E1, E2, W3vT, W3e1T, W3e2T, W3e3T, b3.reshape(1, 128))


# speedup vs baseline: 6.8415x; 1.0076x over previous
import functools

import jax
import jax.numpy as jnp
from jax import lax
from jax.experimental import pallas as pl
from jax.experimental.pallas import tpu as pltpu
from jax.experimental.pallas import tpu_sc as plsc

B = 16384
D3 = 256
_GATHER_CHUNK = 128


def _leaky(x):
    return jnp.where(x > 0, x, 0.01 * x)


def _sc_gather(table, idx2d):
    info = plsc.get_sparse_core_info()
    nw = info.num_cores * info.num_subcores
    b_per_w = B // nw
    n_chunks = b_per_w // _GATHER_CHUNK

    mesh = plsc.VectorSubcoreMesh(core_axis_name="c", subcore_axis_name="s")

    @functools.partial(
        pl.kernel,
        mesh=mesh,
        out_type=jax.ShapeDtypeStruct((B, D3), jnp.float32),
        scratch_types=[
            pltpu.VMEM((n_chunks, _GATHER_CHUNK), jnp.int32),
            pltpu.VMEM((_GATHER_CHUNK, D3), jnp.float32),
            pltpu.VMEM((_GATHER_CHUNK, D3), jnp.float32),
            pltpu.SemaphoreType.DMA,
            pltpu.SemaphoreType.DMA,
            pltpu.SemaphoreType.DMA,
            pltpu.SemaphoreType.DMA,
        ],
    )
    def gather_k(table_hbm, idx_hbm, out_hbm, idx_v, rows_a, rows_b,
                 gsem_a, gsem_b, osem_a, osem_b):
        wid = lax.axis_index("s") * info.num_cores + lax.axis_index("c")
        base = wid * b_per_w
        rows = (rows_a, rows_b)
        gsem = (gsem_a, gsem_b)
        osem = (osem_a, osem_b)
        pltpu.sync_copy(idx_hbm.at[pl.ds(wid * n_chunks, n_chunks)], idx_v)
        out_copies = [None] * n_chunks
        prev = None
        for c in range(n_chunks):
            if c >= 2:
                out_copies[c - 2].wait()
            g = pltpu.async_copy(table_hbm.at[idx_v.at[c]], rows[c % 2],
                                 gsem[c % 2])
            if prev is not None:
                pc, pg = prev
                pg.wait()
                out_copies[pc] = pltpu.async_copy(
                    rows[pc % 2],
                    out_hbm.at[pl.ds(base + pc * _GATHER_CHUNK, _GATHER_CHUNK)],
                    osem[pc % 2])
            prev = (c, g)
        pc, pg = prev
        pg.wait()
        out_copies[pc] = pltpu.async_copy(
            rows[pc % 2],
            out_hbm.at[pl.ds(base + pc * _GATHER_CHUNK, _GATHER_CHUNK)],
            osem[pc % 2])
        out_copies[n_chunks - 2].wait()
        out_copies[n_chunks - 1].wait()

    return gather_k(table, idx2d)


def _tc_body(num_ref, c1_ref, c2_ref, e3_ref,
             W1T_ref, b1_ref, W2T_ref, b2_ref,
             E1_ref, E2_ref, W3vT_ref, W3e1T_ref, W3e2T_ref, W3e3T_ref,
             b3_ref, out_ref):
    blk = num_ref.shape[0]
    x = num_ref[...]
    v = _leaky(jnp.dot(x, W1T_ref[...], preferred_element_type=jnp.float32)
               + b1_ref[...])
    v = _leaky(jnp.dot(v, W2T_ref[...], preferred_element_type=jnp.float32)
               + b2_ref[...])
    acc = jnp.dot(v, W3vT_ref[...], preferred_element_type=jnp.float32)
    acc += jnp.dot(e3_ref[...], W3e3T_ref[...],
                   preferred_element_type=jnp.float32)
    oh1 = (c1_ref[...] == lax.broadcasted_iota(jnp.int32, (blk, 4), 1)
           ).astype(jnp.float32)
    e1 = jnp.dot(oh1, E1_ref[...], preferred_element_type=jnp.float32)
    acc += jnp.dot(e1, W3e1T_ref[...], preferred_element_type=jnp.float32)
    oh2 = (c2_ref[...] == lax.broadcasted_iota(jnp.int32, (blk, 5), 1)
           ).astype(jnp.float32)
    e2 = jnp.dot(oh2, E2_ref[...], preferred_element_type=jnp.float32)
    acc += jnp.dot(e2, W3e2T_ref[...], preferred_element_type=jnp.float32)
    acc += b3_ref[...]
    out_ref[...] = _leaky(acc)


def _tc_fused(numeric, c1, c2, e3, W1T, b1, W2T, b2,
              E1, E2, W3vT, W3e1T, W3e2T, W3e3T, b3, blk=1024):
    grid = B // blk

    def full(shape):
        return pl.BlockSpec(shape, lambda i: (0, 0))

    return pl.pallas_call(
        _tc_body,
        grid=(grid,),
        in_specs=[
            pl.BlockSpec((blk, 3), lambda i: (i, 0)),
            pl.BlockSpec((blk, 1), lambda i: (i, 0)),
            pl.BlockSpec((blk, 1), lambda i: (i, 0)),
            pl.BlockSpec((blk, D3), lambda i: (i, 0)),
            full((3, 64)),
            full((1, 64)),
            full((64, 128)),
            full((1, 128)),
            full((4, 16)),
            full((5, 25)),
            full((128, 128)),
            full((16, 128)),
            full((25, 128)),
            full((D3, 128)),
            full((1, 128)),
        ],
        out_specs=pl.BlockSpec((blk, 128), lambda i: (i, 0)),
        out_shape=jax.ShapeDtypeStruct((B, 128), jnp.float32),
    )(numeric, c1, c2, e3, W1T, b1, W2T, b2,
      E1, E2, W3vT, W3e1T, W3e2T, W3e3T, b3)


def kernel(numeric, cat1, cat2, cat3, W1, b1, W2, b2, E1, E2, E3, W3, b3):
    idx2d = cat3.reshape(B // _GATHER_CHUNK, _GATHER_CHUNK).astype(jnp.int32)
    e3 = _sc_gather(E3, idx2d)

    c1 = cat1.reshape(B, 1).astype(jnp.int32)
    c2 = cat2.reshape(B, 1).astype(jnp.int32)
    W3vT = W3[:, 0:128].T
    W3e1T = W3[:, 128:144].T
    W3e2T = W3[:, 144:169].T
    W3e3T = W3[:, 169:425].T
    return _tc_fused(numeric, c1, c2, e3,
                     W1.T, b1.reshape(1, 64), W2.T, b2.reshape(1, 128),
                     E1, E2, W3vT, W3e1T, W3e2T, W3e3T, b3.reshape(1, 128))
